# baseline (speedup 1.0000x reference)
import functools

import jax
import jax.numpy as jnp
from jax import lax
from jax.experimental import pallas as pl
from jax.experimental.pallas import tpu as pltpu
from jax.experimental.pallas import tpu_sc as plsc

BATCH = 16384
SC_BATCH = 3072
TC_BATCH = BATCH - SC_BATCH
TCB = 1024
NWORKERS = 32
ROWS_PER = SC_BATCH // NWORKERS
GROUPS = ROWS_PER // 16


def _tt_body(x_hbm, t1_hbm, t2_hbm, t3_hbm, out_hbm, xv, t1v, t2v, t3v, outv):
    wid = lax.axis_index("s") * 2 + lax.axis_index("c")
    base = wid * ROWS_PER

    pltpu.sync_copy(x_hbm.at[pl.ds(base, ROWS_PER)], xv)
    pltpu.sync_copy(t1_hbm, t1v)
    pltpu.sync_copy(t2_hbm, t2v)
    pltpu.sync_copy(t3_hbm, t3v)

    iota = lax.iota(jnp.int32, 16)

    @plsc.parallel_loop(0, GROUPS, 1, unroll=2)
    def group(g):
        xvec = xv[pl.ds(g * 16, 16)]
        i1 = xvec // 10000
        r = xvec - i1 * 10000
        i2 = r // 100
        i3 = r - i2 * 100
        rows = g * 16 + iota
        j1p = [i1 * 32 + p for p in range(8)]
        j2cq = [[i2 * 32 + (c * 8 + q) for q in range(8)] for c in range(4)]
        j3d = [i3 * 4 + d for d in range(4)]

        def g1(a, p):
            return plsc.load_gather(t1v.at[pl.ds(a * 8, 3200 - a * 8)], [j1p[p]])

        def g2(p, c, q):
            return plsc.load_gather(
                t2v.at[pl.ds(p * 3200, 25600 - p * 3200)], [j2cq[c][q]])

        def g3(q, d):
            return plsc.load_gather(t3v.at[pl.ds(q * 400, 3200 - q * 400)], [j3d[d]])

        a1 = [[g1(a, p) for p in range(8)] for a in range(4)]
        a3 = [[g3(q, d) for d in range(4)] for q in range(8)]
        for c in range(4):
            acc = [[None] * 4 for _ in range(4)]
            for q in range(8):
                a2 = [g2(p, c, q) for p in range(8)]
                t = []
                for a in range(4):
                    s = a1[a][0] * a2[0]
                    for p in range(1, 8):
                        s = s + a1[a][p] * a2[p]
                    t.append(s)
                for d in range(4):
                    for a in range(4):
                        prod = t[a] * a3[q][d]
                        acc[a][d] = prod if acc[a][d] is None else acc[a][d] + prod
            for a in range(4):
                for d in range(4):
                    fo = a * 16 + c * 4 + d
                    plsc.store_scatter(
                        outv, [jnp.full((16,), fo, jnp.int32), rows], acc[a][d])

    pltpu.sync_copy(outv, out_hbm.at[wid])


def _tt_call(x, t1, t2, t3):
    mesh = plsc.VectorSubcoreMesh(core_axis_name="c", subcore_axis_name="s")
    f = functools.partial(
        pl.kernel,
        out_type=jax.ShapeDtypeStruct((NWORKERS, 64, ROWS_PER), jnp.float32),
        mesh=mesh,
        compiler_params=pltpu.CompilerParams(needs_layout_passes=False),
        scratch_types=[
            pltpu.VMEM((ROWS_PER,), jnp.int32),
            pltpu.VMEM((3200,), jnp.float32),
            pltpu.VMEM((25600,), jnp.float32),
            pltpu.VMEM((3200,), jnp.float32),
            pltpu.VMEM((64, ROWS_PER), jnp.float32),
        ],
    )(_tt_body)
    return f(x, t1, t2, t3)


def _tc_body(x_ref, g1_ref, g2_ref, g3_ref, out_ref):
    x = x_ref[0]
    i1 = x // 10000
    r = x - i1 * 10000
    i2 = r // 100
    i3 = r - i2 * 100

    def onehot(iv):
        rows = lax.broadcasted_iota(jnp.int32, (100, TCB), 0)
        return (rows == iv).astype(jnp.float32)

    a1t = jnp.dot(g1_ref[...], onehot(i1), preferred_element_type=jnp.float32)
    a2t = jnp.dot(g2_ref[...], onehot(i2), preferred_element_type=jnp.float32)
    a3t = jnp.dot(g3_ref[...], onehot(i3), preferred_element_type=jnp.float32)

    a1r = a1t.reshape(4, 8, TCB)
    a2r = a2t.reshape(8, 4, 8, TCB)
    a3r = a3t.reshape(4, 8, TCB)

    t2 = None
    for p in range(8):
        a1b = a1r[:, p, :].reshape(4, 1, 1, TCB)
        term = a1b * a2r[p][None]
        t2 = term if t2 is None else t2 + term

    term2 = t2[:, :, None, :, :] * a3r[None, None, :, :, :]
    out_ref[...] = jnp.sum(term2, axis=3).reshape(64, TCB)


def _tc_call(x3, g1m, g2m, g3m):
    return pl.pallas_call(
        _tc_body,
        grid=(TC_BATCH // TCB,),
        in_specs=[
            pl.BlockSpec((1, 1, TCB), lambda i: (i, 0, 0)),
            pl.BlockSpec((32, 100), lambda i: (0, 0)),
            pl.BlockSpec((256, 100), lambda i: (0, 0)),
            pl.BlockSpec((32, 100), lambda i: (0, 0)),
        ],
        out_specs=pl.BlockSpec((64, TCB), lambda i: (0, i)),
        out_shape=jax.ShapeDtypeStruct((64, TC_BATCH), jnp.float32),
    )(x3, g1m, g2m, g3m)


def kernel(x, G1, G2, G3):
    xshape = list(x.shape)
    xf = x.reshape(-1).astype(jnp.int32)

    out_sc = _tt_call(xf[:SC_BATCH], G1.reshape(-1), G2.reshape(-1),
                      G3.reshape(-1))
    sc64 = jnp.transpose(out_sc, (1, 0, 2)).reshape(64, SC_BATCH)

    f1 = jnp.transpose(G1[0], (1, 2, 0))
    f2 = jnp.transpose(G2, (0, 2, 3, 1))
    f3d = jnp.transpose(G3[..., 0], (2, 0, 1))
    x3 = xf[SC_BATCH:].reshape(TC_BATCH // TCB, 1, TCB)
    r100 = lambda m: m.reshape(-1, 100)
    tc64 = _tc_call(x3, r100(f1), r100(f2), r100(f3d))

    merged = jnp.concatenate([sc64, tc64], axis=1)
    rows = jnp.transpose(merged, (1, 0))
    return rows.reshape(xshape + [64])


# speedup vs baseline: 1.1898x; 1.1898x over previous
import functools

import jax
import jax.numpy as jnp
from jax import lax
from jax.experimental import pallas as pl
from jax.experimental.pallas import tpu as pltpu
from jax.experimental.pallas import tpu_sc as plsc

BATCH = 16384
SC_BATCH = 3072
TC_BATCH = BATCH - SC_BATCH
TCB = 1024
NWORKERS = 32
ROWS_PER = SC_BATCH // NWORKERS
GROUPS = ROWS_PER // 16


def _tt_body(x_hbm, t1_hbm, t2_hbm, t3_hbm, out_hbm, xv, t1v, t2v, t3v, outv):
    wid = lax.axis_index("s") * 2 + lax.axis_index("c")
    base = wid * ROWS_PER

    pltpu.sync_copy(x_hbm.at[pl.ds(base, ROWS_PER)], xv)
    pltpu.sync_copy(t1_hbm, t1v)
    pltpu.sync_copy(t2_hbm, t2v)
    pltpu.sync_copy(t3_hbm, t3v)

    iota = lax.iota(jnp.int32, 16)

    @plsc.parallel_loop(0, GROUPS, 1, unroll=2)
    def group(g):
        xvec = xv[pl.ds(g * 16, 16)]
        i1 = xvec // 10000
        r = xvec - i1 * 10000
        i2 = r // 100
        i3 = r - i2 * 100
        rows = g * 16 + iota

        j3 = i3 * 4

        def g1(f):
            return plsc.load_gather(t1v, [i1 + f * 100])

        def g2(f):
            return plsc.load_gather(t2v, [i2 + f * 100])

        def g3(q, d):
            return plsc.load_gather(t3v, [j3 + (q * 400 + d)])

        a1 = [[g1(a * 8 + p) for p in range(8)] for a in range(4)]
        a3 = [[g3(q, d) for d in range(4)] for q in range(8)]
        for c in range(4):
            acc = [[None] * 4 for _ in range(4)]
            for q in range(8):
                a2 = [g2((p * 4 + c) * 8 + q) for p in range(8)]
                t = []
                for a in range(4):
                    s = a1[a][0] * a2[0]
                    for p in range(1, 8):
                        s = s + a1[a][p] * a2[p]
                    t.append(s)
                for d in range(4):
                    for a in range(4):
                        prod = t[a] * a3[q][d]
                        acc[a][d] = prod if acc[a][d] is None else acc[a][d] + prod
            for a in range(4):
                for d in range(4):
                    fo = a * 16 + c * 4 + d
                    plsc.store_scatter(
                        outv, [jnp.full((16,), fo, jnp.int32), rows], acc[a][d])

    pltpu.sync_copy(outv, out_hbm.at[wid])


def _tt_call(x, t1, t2, t3):
    mesh = plsc.VectorSubcoreMesh(core_axis_name="c", subcore_axis_name="s")
    f = functools.partial(
        pl.kernel,
        out_type=jax.ShapeDtypeStruct((NWORKERS, 64, ROWS_PER), jnp.float32),
        mesh=mesh,
        compiler_params=pltpu.CompilerParams(needs_layout_passes=False),
        scratch_types=[
            pltpu.VMEM((ROWS_PER,), jnp.int32),
            pltpu.VMEM((3200,), jnp.float32),
            pltpu.VMEM((25600,), jnp.float32),
            pltpu.VMEM((3200,), jnp.float32),
            pltpu.VMEM((64, ROWS_PER), jnp.float32),
        ],
    )(_tt_body)
    return f(x, t1, t2, t3)


def _tc_body(x_ref, g1_ref, g2_ref, g3_ref, out_ref):
    x = x_ref[0]
    i1 = x // 10000
    r = x - i1 * 10000
    i2 = r // 100
    i3 = r - i2 * 100

    def onehot(iv):
        rows = lax.broadcasted_iota(jnp.int32, (100, TCB), 0)
        return (rows == iv).astype(jnp.float32)

    a1t = jnp.dot(g1_ref[...], onehot(i1), preferred_element_type=jnp.float32)
    a2t = jnp.dot(g2_ref[...], onehot(i2), preferred_element_type=jnp.float32)
    a3t = jnp.dot(g3_ref[...], onehot(i3), preferred_element_type=jnp.float32)

    a1r = a1t.reshape(4, 8, TCB)
    a2r = a2t.reshape(8, 4, 8, TCB)
    a3r = a3t.reshape(4, 8, TCB)

    t2 = None
    for p in range(8):
        a1b = a1r[:, p, :].reshape(4, 1, 1, TCB)
        term = a1b * a2r[p][None]
        t2 = term if t2 is None else t2 + term

    term2 = t2[:, :, None, :, :] * a3r[None, None, :, :, :]
    out_ref[...] = jnp.sum(term2, axis=3).reshape(64, TCB)


def _tc_call(x3, g1m, g2m, g3m):
    return pl.pallas_call(
        _tc_body,
        grid=(TC_BATCH // TCB,),
        in_specs=[
            pl.BlockSpec((1, 1, TCB), lambda i: (i + SC_BATCH // TCB, 0, 0)),
            pl.BlockSpec((32, 100), lambda i: (0, 0)),
            pl.BlockSpec((256, 100), lambda i: (0, 0)),
            pl.BlockSpec((32, 100), lambda i: (0, 0)),
        ],
        out_specs=pl.BlockSpec((64, TCB), lambda i: (0, i)),
        out_shape=jax.ShapeDtypeStruct((64, TC_BATCH), jnp.float32),
    )(x3, g1m, g2m, g3m)


def kernel(x, G1, G2, G3):
    xshape = list(x.shape)
    xf = x.reshape(-1).astype(jnp.int32)

    f1 = jnp.transpose(G1[0], (1, 2, 0))
    f2 = jnp.transpose(G2, (0, 2, 3, 1))
    f3d = jnp.transpose(G3[..., 0], (2, 0, 1))

    out_sc = _tt_call(xf, f1.reshape(-1), f2.reshape(-1), G3.reshape(-1))
    sc64 = jnp.transpose(out_sc, (1, 0, 2)).reshape(64, SC_BATCH)

    x3 = xf.reshape(BATCH // TCB, 1, TCB)
    r100 = lambda m: m.reshape(-1, 100)
    tc64 = _tc_call(x3, r100(f1), r100(f2), r100(f3d))

    merged = jnp.concatenate([sc64, tc64], axis=1)
    rows = jnp.transpose(merged, (1, 0))
    return rows.reshape(xshape + [64])


# speedup vs baseline: 1.3295x; 1.1174x over previous
import functools

import jax
import jax.numpy as jnp
from jax import lax
from jax.experimental import pallas as pl
from jax.experimental.pallas import tpu as pltpu
from jax.experimental.pallas import tpu_sc as plsc

BATCH = 16384
SC_BATCH = 2048
TC_BATCH = BATCH - SC_BATCH
TCB = 1024
NWORKERS = 32
ROWS_PER = SC_BATCH // NWORKERS
GROUPS = ROWS_PER // 16


def _tt_body(x_hbm, t1_hbm, t2_hbm, t3_hbm, out_hbm, xv, t1v, t2v, t3v, outv):
    wid = lax.axis_index("s") * 2 + lax.axis_index("c")
    base = wid * ROWS_PER

    pltpu.sync_copy(x_hbm.at[pl.ds(base, ROWS_PER)], xv)
    pltpu.sync_copy(t1_hbm, t1v)
    pltpu.sync_copy(t2_hbm, t2v)
    pltpu.sync_copy(t3_hbm, t3v)

    iota = lax.iota(jnp.int32, 16)

    @plsc.parallel_loop(0, GROUPS, 1, unroll=2)
    def group(g):
        xvec = xv[pl.ds(g * 16, 16)]
        i1 = xvec // 10000
        r = xvec - i1 * 10000
        i2 = r // 100
        i3 = r - i2 * 100
        rows = g * 16 + iota

        j3 = i3 * 4

        def g1(f):
            return plsc.load_gather(t1v, [i1 + f * 100])

        def g2(f):
            return plsc.load_gather(t2v, [i2 + f * 100])

        def g3(q, d):
            return plsc.load_gather(t3v, [j3 + (q * 400 + d)])

        a1 = [[g1(a * 8 + p) for p in range(8)] for a in range(4)]
        a3 = [[g3(q, d) for d in range(4)] for q in range(8)]
        for c in range(4):
            acc = [[None] * 4 for _ in range(4)]
            for q in range(8):
                a2 = [g2((p * 4 + c) * 8 + q) for p in range(8)]
                t = []
                for a in range(4):
                    s = a1[a][0] * a2[0]
                    for p in range(1, 8):
                        s = s + a1[a][p] * a2[p]
                    t.append(s)
                for d in range(4):
                    for a in range(4):
                        prod = t[a] * a3[q][d]
                        acc[a][d] = prod if acc[a][d] is None else acc[a][d] + prod
            for a in range(4):
                for d in range(4):
                    fo = a * 16 + c * 4 + d
                    plsc.store_scatter(
                        outv, [jnp.full((16,), fo, jnp.int32), rows], acc[a][d])

    pltpu.sync_copy(outv, out_hbm.at[wid])


def _tt_call(x, t1, t2, t3):
    mesh = plsc.VectorSubcoreMesh(core_axis_name="c", subcore_axis_name="s")
    f = functools.partial(
        pl.kernel,
        out_type=jax.ShapeDtypeStruct((NWORKERS, 64, ROWS_PER), jnp.float32),
        mesh=mesh,
        compiler_params=pltpu.CompilerParams(needs_layout_passes=False),
        scratch_types=[
            pltpu.VMEM((ROWS_PER,), jnp.int32),
            pltpu.VMEM((3200,), jnp.float32),
            pltpu.VMEM((25600,), jnp.float32),
            pltpu.VMEM((3200,), jnp.float32),
            pltpu.VMEM((64, ROWS_PER), jnp.float32),
        ],
    )(_tt_body)
    return f(x, t1, t2, t3)


def _tc_body(x_ref, g1_ref, g2_ref, g3_ref, out_ref):
    x = x_ref[0]
    i1 = x // 10000
    r = x - i1 * 10000
    i2 = r // 100
    i3 = r - i2 * 100

    def onehot(iv):
        rows = lax.broadcasted_iota(jnp.int32, (100, TCB), 0)
        return (rows == iv).astype(jnp.float32)

    a1t = jnp.dot(g1_ref[...], onehot(i1), preferred_element_type=jnp.float32)
    a2t = jnp.dot(g2_ref[...], onehot(i2), preferred_element_type=jnp.float32)
    a3t = jnp.dot(g3_ref[...], onehot(i3), preferred_element_type=jnp.float32)

    a1r = a1t.reshape(4, 8, TCB)
    a2r = a2t.reshape(8, 4, 8, TCB)
    a3r = a3t.reshape(4, 8, TCB)

    t2 = None
    for p in range(8):
        a1b = a1r[:, p, :].reshape(4, 1, 1, TCB)
        term = a1b * a2r[p][None]
        t2 = term if t2 is None else t2 + term

    term2 = t2[:, :, None, :, :] * a3r[None, None, :, :, :]
    out_ref[...] = jnp.sum(term2, axis=3).reshape(64, TCB)


def _tc_call(x3, g1m, g2m, g3m):
    return pl.pallas_call(
        _tc_body,
        grid=(TC_BATCH // TCB,),
        in_specs=[
            pl.BlockSpec((1, 1, TCB), lambda i: (i + SC_BATCH // TCB, 0, 0)),
            pl.BlockSpec((32, 100), lambda i: (0, 0)),
            pl.BlockSpec((256, 100), lambda i: (0, 0)),
            pl.BlockSpec((32, 100), lambda i: (0, 0)),
        ],
        out_specs=pl.BlockSpec((64, TCB), lambda i: (0, i)),
        out_shape=jax.ShapeDtypeStruct((64, TC_BATCH), jnp.float32),
    )(x3, g1m, g2m, g3m)


def kernel(x, G1, G2, G3):
    xshape = list(x.shape)
    xf = x.reshape(-1).astype(jnp.int32)

    f1 = jnp.transpose(G1[0], (1, 2, 0))
    f2 = jnp.transpose(G2, (0, 2, 3, 1))
    f3d = jnp.transpose(G3[..., 0], (2, 0, 1))

    out_sc = _tt_call(xf, f1.reshape(-1), f2.reshape(-1), G3.reshape(-1))
    sc64 = jnp.transpose(out_sc, (1, 0, 2)).reshape(64, SC_BATCH)

    x3 = xf.reshape(BATCH // TCB, 1, TCB)
    r100 = lambda m: m.reshape(-1, 100)
    tc64 = _tc_call(x3, r100(f1), r100(f2), r100(f3d))

    merged = jnp.concatenate([sc64, tc64], axis=1)
    rows = jnp.transpose(merged, (1, 0))
    return rows.reshape(xshape + [64])


# speedup vs baseline: 1.3312x; 1.0013x over previous
import functools

import jax
import jax.numpy as jnp
from jax import lax
from jax.experimental import pallas as pl
from jax.experimental.pallas import tpu as pltpu
from jax.experimental.pallas import tpu_sc as plsc

BATCH = 16384
SC_BATCH = 1024
TC_BATCH = BATCH - SC_BATCH
TCB = 1024
NWORKERS = 32
ROWS_PER = SC_BATCH // NWORKERS
GROUPS = ROWS_PER // 16


def _tt_body(x_hbm, t1_hbm, t2_hbm, t3_hbm, out_hbm, xv, t1v, t2v, t3v, outv):
    wid = lax.axis_index("s") * 2 + lax.axis_index("c")
    base = wid * ROWS_PER

    pltpu.sync_copy(x_hbm.at[pl.ds(base, ROWS_PER)], xv)
    pltpu.sync_copy(t1_hbm, t1v)
    pltpu.sync_copy(t2_hbm, t2v)
    pltpu.sync_copy(t3_hbm, t3v)

    iota = lax.iota(jnp.int32, 16)

    @plsc.parallel_loop(0, GROUPS, 1, unroll=2)
    def group(g):
        xvec = xv[pl.ds(g * 16, 16)]
        i1 = xvec // 10000
        r = xvec - i1 * 10000
        i2 = r // 100
        i3 = r - i2 * 100
        rows = g * 16 + iota

        j3 = i3 * 4

        def g1(f):
            return plsc.load_gather(t1v, [i1 + f * 100])

        def g2(f):
            return plsc.load_gather(t2v, [i2 + f * 100])

        def g3(q, d):
            return plsc.load_gather(t3v, [j3 + (q * 400 + d)])

        a1 = [[g1(a * 8 + p) for p in range(8)] for a in range(4)]
        a3 = [[g3(q, d) for d in range(4)] for q in range(8)]
        for c in range(4):
            acc = [[None] * 4 for _ in range(4)]
            for q in range(8):
                a2 = [g2((p * 4 + c) * 8 + q) for p in range(8)]
                t = []
                for a in range(4):
                    s = a1[a][0] * a2[0]
                    for p in range(1, 8):
                        s = s + a1[a][p] * a2[p]
                    t.append(s)
                for d in range(4):
                    for a in range(4):
                        prod = t[a] * a3[q][d]
                        acc[a][d] = prod if acc[a][d] is None else acc[a][d] + prod
            for a in range(4):
                for d in range(4):
                    fo = a * 16 + c * 4 + d
                    plsc.store_scatter(
                        outv, [jnp.full((16,), fo, jnp.int32), rows], acc[a][d])

    pltpu.sync_copy(outv, out_hbm.at[wid])


def _tt_call(x, t1, t2, t3):
    mesh = plsc.VectorSubcoreMesh(core_axis_name="c", subcore_axis_name="s")
    f = functools.partial(
        pl.kernel,
        out_type=jax.ShapeDtypeStruct((NWORKERS, 64, ROWS_PER), jnp.float32),
        mesh=mesh,
        compiler_params=pltpu.CompilerParams(needs_layout_passes=False),
        scratch_types=[
            pltpu.VMEM((ROWS_PER,), jnp.int32),
            pltpu.VMEM((3200,), jnp.float32),
            pltpu.VMEM((25600,), jnp.float32),
            pltpu.VMEM((3200,), jnp.float32),
            pltpu.VMEM((64, ROWS_PER), jnp.float32),
        ],
    )(_tt_body)
    return f(x, t1, t2, t3)


def _tc_body(x_ref, g1_ref, g2_ref, g3_ref, out_ref):
    x = x_ref[0]
    i1 = x // 10000
    r = x - i1 * 10000
    i2 = r // 100
    i3 = r - i2 * 100

    def onehot(iv):
        rows = lax.broadcasted_iota(jnp.int32, (100, TCB), 0)
        return (rows == iv).astype(jnp.float32)

    a1t = jnp.dot(g1_ref[...], onehot(i1), preferred_element_type=jnp.float32)
    a2t = jnp.dot(g2_ref[...], onehot(i2), preferred_element_type=jnp.float32)
    a3t = jnp.dot(g3_ref[...], onehot(i3), preferred_element_type=jnp.float32)

    a1r = a1t.reshape(4, 8, TCB)
    a2r = a2t.reshape(8, 4, 8, TCB)
    a3r = a3t.reshape(4, 8, TCB)

    t2 = None
    for p in range(8):
        a1b = a1r[:, p, :].reshape(4, 1, 1, TCB)
        term = a1b * a2r[p][None]
        t2 = term if t2 is None else t2 + term

    term2 = t2[:, :, None, :, :] * a3r[None, None, :, :, :]
    out_ref[...] = jnp.sum(term2, axis=3).reshape(64, TCB)


def _tc_call(x3, g1m, g2m, g3m):
    return pl.pallas_call(
        _tc_body,
        grid=(TC_BATCH // TCB,),
        in_specs=[
            pl.BlockSpec((1, 1, TCB), lambda i: (i + SC_BATCH // TCB, 0, 0)),
            pl.BlockSpec((32, 100), lambda i: (0, 0)),
            pl.BlockSpec((256, 100), lambda i: (0, 0)),
            pl.BlockSpec((32, 100), lambda i: (0, 0)),
        ],
        out_specs=pl.BlockSpec((64, TCB), lambda i: (0, i)),
        out_shape=jax.ShapeDtypeStruct((64, TC_BATCH), jnp.float32),
    )(x3, g1m, g2m, g3m)


def kernel(x, G1, G2, G3):
    xshape = list(x.shape)
    xf = x.reshape(-1).astype(jnp.int32)

    f1 = jnp.transpose(G1[0], (1, 2, 0))
    f2 = jnp.transpose(G2, (0, 2, 3, 1))
    f3d = jnp.transpose(G3[..., 0], (2, 0, 1))

    out_sc = _tt_call(xf, f1.reshape(-1), f2.reshape(-1), G3.reshape(-1))
    sc64 = jnp.transpose(out_sc, (1, 0, 2)).reshape(64, SC_BATCH)

    x3 = xf.reshape(BATCH // TCB, 1, TCB)
    r100 = lambda m: m.reshape(-1, 100)
    tc64 = _tc_call(x3, r100(f1), r100(f2), r100(f3d))

    merged = jnp.concatenate([sc64, tc64], axis=1)
    rows = jnp.transpose(merged, (1, 0))
    return rows.reshape(xshape + [64])
